# baseline (speedup 1.0000x reference)
import functools

import jax
import jax.numpy as jnp
from jax import lax
from jax.experimental import pallas as pl
from jax.experimental.pallas import tpu as pltpu
from jax.experimental.pallas import tpu_sc as plsc

B = 16384
L = 50
V = 100000
D = 128
H = 1024

LP = 64
NC, NS = 2, 16
NW = NC * NS
NT = 2 * B
TW = NT // NW
RND = 4
NROUND = TW // RND

_EPS_BN = 1e-5


def _sc_pool_body(idx_hbm, table_hbm, out_hbm, idxbuf, rmbuf, rowsbuf, outbuf,
                  sem0, sem1):
    wid = lax.axis_index("s") * NC + lax.axis_index("c")
    t0 = wid * TW
    pltpu.sync_copy(idx_hbm.at[pl.ds(t0 * LP, TW * LP)], idxbuf)

    def round_body(r, carry):
        base = r * RND * LP
        for t in range(RND):
            for k in range(LP // 16):
                off = t * LP + k * 16
                iv = idxbuf[pl.ds(base + off, 16)]
                rmbuf[pl.ds(off, 16)] = jnp.where(iv != 0, iv, V)
        cp0 = pltpu.async_copy(table_hbm.at[rmbuf.at[pl.ds(0, 128)]],
                               rowsbuf.at[pl.ds(0, 128)], sem0)
        cp1 = pltpu.async_copy(table_hbm.at[rmbuf.at[pl.ds(128, 128)]],
                               rowsbuf.at[pl.ds(128, 128)], sem1)
        cp0.wait()
        cp1.wait()
        for t in range(RND):
            accs = tuple(jnp.zeros((16,), jnp.float32) for _ in range(D // 16))

            def sum_body(j, accs, _t=t):
                out = list(accs)
                for u in range(4):
                    row = _t * LP + j * 4 + u
                    for cc in range(D // 16):
                        out[cc] = out[cc] + rowsbuf[row, pl.ds(cc * 16, 16)]
                return tuple(out)

            accs = lax.fori_loop(0, LP // 4, sum_body, accs)
            for cc in range(D // 16):
                outbuf[t, pl.ds(cc * 16, 16)] = accs[cc]
        pltpu.sync_copy(outbuf, out_hbm.at[pl.ds(t0 + r * RND, RND)])
        return carry

    lax.fori_loop(0, NROUND, round_body, 0)


@functools.lru_cache(maxsize=None)
def _sc_pool():
    return functools.partial(
        pl.kernel,
        out_type=jax.ShapeDtypeStruct((NT, D), jnp.float32),
        mesh=plsc.VectorSubcoreMesh(core_axis_name="c", subcore_axis_name="s",
                                    num_cores=NC, num_subcores=NS),
        scratch_types=[
            pltpu.VMEM((TW * LP,), jnp.int32),
            pltpu.VMEM((RND * LP,), jnp.int32),
            pltpu.VMEM((RND * LP, D), jnp.float32),
            pltpu.VMEM((RND, D), jnp.float32),
            pltpu.SemaphoreType.DMA,
            pltpu.SemaphoreType.DMA,
        ],
    )(_sc_pool_body)


RB = 512
GRID = B // RB


def _k1_body(ps_ref, pd_ref, si_ref, di_ref, w1_ref, b1_ref, h_ref, st_ref):
    i = pl.program_id(0)
    cs = jnp.sum((si_ref[...] != 0).astype(jnp.float32), axis=1, keepdims=True)
    cd = jnp.sum((di_ref[...] != 0).astype(jnp.float32), axis=1, keepdims=True)
    s = ps_ref[...] / jnp.maximum(cs, 1e-9)
    dd = pd_ref[...] / jnp.maximum(cd, 1e-9)
    h = jnp.dot(s, w1_ref[0:D, :], preferred_element_type=jnp.float32)
    h = h + jnp.dot(dd, w1_ref[D:2 * D, :],
                    preferred_element_type=jnp.float32)
    h = jnp.maximum(h + b1_ref[...], 0.0)
    h_ref[...] = h
    s = jnp.sum(h, axis=0, keepdims=True)
    q = jnp.sum(h * h, axis=0, keepdims=True)
    sq = jnp.concatenate([s, q], axis=0)

    @pl.when(i == 0)
    def _():
        st_ref[...] = jnp.zeros_like(st_ref)

    st_ref[...] += sq


def _bn_scale_shift(st_ref, g_ref, be_ref):
    mu = st_ref[0:1, :] * (1.0 / B)
    var = st_ref[1:2, :] * (1.0 / B) - mu * mu
    scale = g_ref[...] * jax.lax.rsqrt(var + _EPS_BN)
    shift = be_ref[...] - mu * scale
    return scale, shift


def _k2_body(h_ref, st_ref, g_ref, be_ref, w_ref, b_ref, o_ref, st2_ref):
    i = pl.program_id(0)
    scale, shift = _bn_scale_shift(st_ref, g_ref, be_ref)
    u = h_ref[...] * scale + shift
    h = jnp.dot(u, w_ref[...], preferred_element_type=jnp.float32)
    h = jnp.maximum(h + b_ref[...], 0.0)
    o_ref[...] = h
    s = jnp.sum(h, axis=0, keepdims=True)
    q = jnp.sum(h * h, axis=0, keepdims=True)
    sq = jnp.concatenate([s, q], axis=0)

    @pl.when(i == 0)
    def _():
        st2_ref[...] = jnp.zeros_like(st2_ref)

    st2_ref[...] += sq


def _k3_body(h_ref, st_ref, g_ref, be_ref, w3_ref, b3_ref, w4_ref, b4_ref,
             o_ref):
    scale, shift = _bn_scale_shift(st_ref, g_ref, be_ref)
    u = h_ref[...] * scale + shift
    h = jnp.dot(u, w3_ref[...], preferred_element_type=jnp.float32)
    h = jnp.maximum(h + b3_ref[...], 0.0)
    z = jnp.sum(h * w4_ref[...], axis=1, keepdims=True) + b4_ref[...]
    o_ref[...] = jax.nn.sigmoid(z)


def _row_spec(ncols):
    return pl.BlockSpec((RB, ncols), lambda i: (i, 0))


def _full_spec(shape):
    return pl.BlockSpec(shape, lambda i: tuple(0 for _ in shape))


def kernel(source, destination, emb, W1, b1, g1, be1, W2, b2, g2, be2, W3, b3,
           W4, b4):
    table = jnp.concatenate(
        [emb, jnp.zeros((8, D), jnp.float32)], axis=0)
    idx = jnp.concatenate([source, destination], axis=0)
    idx = jnp.pad(idx, ((0, 0), (0, LP - L)))
    idx_flat = idx.reshape(-1)

    pooled = _sc_pool()(idx_flat, table)

    b1r = b1.reshape(1, H)
    b2r = b2.reshape(1, H)
    b3r = b3.reshape(1, H)
    g1r = g1.reshape(1, H)
    g2r = g2.reshape(1, H)
    be1r = be1.reshape(1, H)
    be2r = be2.reshape(1, H)
    w4r = W4.reshape(1, H)
    b4r = b4.reshape(1, 1)

    h1, st1 = pl.pallas_call(
        _k1_body,
        grid=(GRID,),
        in_specs=[
            pl.BlockSpec((RB, D), lambda i: (i, 0)),
            pl.BlockSpec((RB, D), lambda i: (i + GRID, 0)),
            pl.BlockSpec((RB, L), lambda i: (i, 0)),
            pl.BlockSpec((RB, L), lambda i: (i, 0)),
            _full_spec((2 * D, H)),
            _full_spec((1, H)),
        ],
        out_specs=[_row_spec(H), _full_spec((2, H))],
        out_shape=[
            jax.ShapeDtypeStruct((B, H), jnp.float32),
            jax.ShapeDtypeStruct((2, H), jnp.float32),
        ],
    )(pooled, pooled, source, destination, W1, b1r)

    h2, st2 = pl.pallas_call(
        _k2_body,
        grid=(GRID,),
        in_specs=[
            _row_spec(H),
            _full_spec((2, H)),
            _full_spec((1, H)),
            _full_spec((1, H)),
            _full_spec((H, H)),
            _full_spec((1, H)),
        ],
        out_specs=[_row_spec(H), _full_spec((2, H))],
        out_shape=[
            jax.ShapeDtypeStruct((B, H), jnp.float32),
            jax.ShapeDtypeStruct((2, H), jnp.float32),
        ],
    )(h1, st1, g1r, be1r, W2, b2r)

    out = pl.pallas_call(
        _k3_body,
        grid=(GRID,),
        in_specs=[
            _row_spec(H),
            _full_spec((2, H)),
            _full_spec((1, H)),
            _full_spec((1, H)),
            _full_spec((H, H)),
            _full_spec((1, H)),
            _full_spec((1, H)),
            _full_spec((1, 1)),
        ],
        out_specs=pl.BlockSpec((RB, 1), lambda i: (i, 0)),
        out_shape=jax.ShapeDtypeStruct((B, 1), jnp.float32),
    )(h2, st2, g2r, be2r, W3, b3r, w4r, b4r)

    return out.reshape(B)


# speedup vs baseline: 24.8834x; 24.8834x over previous
import functools

import jax
import jax.numpy as jnp
from jax import lax
from jax.experimental import pallas as pl
from jax.experimental.pallas import tpu as pltpu
from jax.experimental.pallas import tpu_sc as plsc

B = 16384
L = 50
V = 100000
D = 128
H = 1024

LP = 64
NC, NS = 2, 16
NW = NC * NS
NT = 2 * B
TW = NT // NW
RND = 2
NROUND = TW // RND
GL = RND * L

_EPS_BN = 1e-5

_CHUNK_OFFS = (0, 16, 32, L - 16)


def _sc_pool_body(idx_hbm, table_hbm, out_hbm, idxbuf,
                  rm0, rm1, rows0, rows1, ob0, ob1,
                  sg0, sg1, so0, so1):
    wid = lax.axis_index("s") * NC + lax.axis_index("c")
    t0 = wid * TW
    pltpu.sync_copy(idx_hbm.at[pl.ds(t0 * LP, TW * LP)], idxbuf)

    def remap(r, rm):
        base = r * (RND * LP)
        for t in range(RND):
            for off in _CHUNK_OFFS:
                iv = idxbuf[pl.ds(base + t * LP + off, 16)]
                rm[pl.ds(t * L + off, 16)] = jnp.where(iv != 0, iv, V)

    def sum_store(r, rows, ob, sem):
        for t in range(RND):
            accs = tuple(jnp.zeros((16,), jnp.float32) for _ in range(D // 16))

            def sum_body(j, accs, _t=t):
                out = list(accs)
                for u in range(5):
                    row = _t * L + j * 5 + u
                    for cc in range(D // 16):
                        out[cc] = out[cc] + rows[row, pl.ds(cc * 16, 16)]
                return tuple(out)

            accs = lax.fori_loop(0, L // 5, sum_body, accs)
            for cc in range(D // 16):
                ob[t, pl.ds(cc * 16, 16)] = accs[cc]
        pltpu.async_copy(ob, out_hbm.at[pl.ds(t0 + r * RND, RND)], sem)

    remap(0, rm0)
    pltpu.async_copy(table_hbm.at[rm0], rows0, sg0)

    def iter_body(rr, carry):
        r0 = rr * 2
        r1 = r0 + 1
        remap(r1, rm1)
        pltpu.async_copy(table_hbm.at[rm1], rows1, sg1)
        pltpu.make_async_copy(table_hbm.at[rm0], rows0, sg0).wait()

        @pl.when(rr > 0)
        def _():
            pltpu.make_async_copy(ob0, out_hbm.at[pl.ds(t0, RND)], so0).wait()

        sum_store(r0, rows0, ob0, so0)
        rn = jnp.minimum(r0 + 2, NROUND - 1)
        remap(rn, rm0)
        pltpu.async_copy(table_hbm.at[rm0], rows0, sg0)
        pltpu.make_async_copy(table_hbm.at[rm1], rows1, sg1).wait()

        @pl.when(rr > 0)
        def _():
            pltpu.make_async_copy(ob1, out_hbm.at[pl.ds(t0, RND)], so1).wait()

        sum_store(r1, rows1, ob1, so1)
        return carry

    lax.fori_loop(0, NROUND // 2, iter_body, 0)
    pltpu.make_async_copy(table_hbm.at[rm0], rows0, sg0).wait()
    pltpu.make_async_copy(ob0, out_hbm.at[pl.ds(t0, RND)], so0).wait()
    pltpu.make_async_copy(ob1, out_hbm.at[pl.ds(t0, RND)], so1).wait()


@functools.lru_cache(maxsize=None)
def _sc_pool():
    return functools.partial(
        pl.kernel,
        out_type=jax.ShapeDtypeStruct((NT, D), jnp.float32),
        mesh=plsc.VectorSubcoreMesh(core_axis_name="c", subcore_axis_name="s",
                                    num_cores=NC, num_subcores=NS),
        scratch_types=[
            pltpu.VMEM((TW * LP,), jnp.int32),
            pltpu.VMEM((GL,), jnp.int32),
            pltpu.VMEM((GL,), jnp.int32),
            pltpu.VMEM((GL, D), jnp.float32),
            pltpu.VMEM((GL, D), jnp.float32),
            pltpu.VMEM((RND, D), jnp.float32),
            pltpu.VMEM((RND, D), jnp.float32),
            pltpu.SemaphoreType.DMA,
            pltpu.SemaphoreType.DMA,
            pltpu.SemaphoreType.DMA,
            pltpu.SemaphoreType.DMA,
        ],
    )(_sc_pool_body)


RB = 512
GRID = B // RB


def _k1_body(ps_ref, pd_ref, si_ref, di_ref, w1_ref, b1_ref, h_ref, st_ref):
    i = pl.program_id(0)
    cs = jnp.sum((si_ref[...] != 0).astype(jnp.float32), axis=1, keepdims=True)
    cd = jnp.sum((di_ref[...] != 0).astype(jnp.float32), axis=1, keepdims=True)
    s = ps_ref[...] / jnp.maximum(cs, 1e-9)
    dd = pd_ref[...] / jnp.maximum(cd, 1e-9)
    h = jnp.dot(s, w1_ref[0:D, :], preferred_element_type=jnp.float32)
    h = h + jnp.dot(dd, w1_ref[D:2 * D, :],
                    preferred_element_type=jnp.float32)
    h = jnp.maximum(h + b1_ref[...], 0.0)
    h_ref[...] = h
    s = jnp.sum(h, axis=0, keepdims=True)
    q = jnp.sum(h * h, axis=0, keepdims=True)
    sq = jnp.concatenate([s, q], axis=0)

    @pl.when(i == 0)
    def _():
        st_ref[...] = jnp.zeros_like(st_ref)

    st_ref[...] += sq


def _bn_scale_shift(st_ref, g_ref, be_ref):
    mu = st_ref[0:1, :] * (1.0 / B)
    var = st_ref[1:2, :] * (1.0 / B) - mu * mu
    scale = g_ref[...] * jax.lax.rsqrt(var + _EPS_BN)
    shift = be_ref[...] - mu * scale
    return scale, shift


def _k2_body(h_ref, st_ref, g_ref, be_ref, w_ref, b_ref, o_ref, st2_ref):
    i = pl.program_id(0)
    scale, shift = _bn_scale_shift(st_ref, g_ref, be_ref)
    u = h_ref[...] * scale + shift
    h = jnp.dot(u, w_ref[...], preferred_element_type=jnp.float32)
    h = jnp.maximum(h + b_ref[...], 0.0)
    o_ref[...] = h
    s = jnp.sum(h, axis=0, keepdims=True)
    q = jnp.sum(h * h, axis=0, keepdims=True)
    sq = jnp.concatenate([s, q], axis=0)

    @pl.when(i == 0)
    def _():
        st2_ref[...] = jnp.zeros_like(st2_ref)

    st2_ref[...] += sq


def _k3_body(h_ref, st_ref, g_ref, be_ref, w3_ref, b3_ref, w4_ref, b4_ref,
             o_ref):
    scale, shift = _bn_scale_shift(st_ref, g_ref, be_ref)
    u = h_ref[...] * scale + shift
    h = jnp.dot(u, w3_ref[...], preferred_element_type=jnp.float32)
    h = jnp.maximum(h + b3_ref[...], 0.0)
    z = jnp.sum(h * w4_ref[...], axis=1, keepdims=True) + b4_ref[...]
    o_ref[...] = jax.nn.sigmoid(z)


def _row_spec(ncols):
    return pl.BlockSpec((RB, ncols), lambda i: (i, 0))


def _full_spec(shape):
    return pl.BlockSpec(shape, lambda i: tuple(0 for _ in shape))


def kernel(source, destination, emb, W1, b1, g1, be1, W2, b2, g2, be2, W3, b3,
           W4, b4):
    table = jnp.concatenate(
        [emb, jnp.zeros((8, D), jnp.float32)], axis=0)
    idx = jnp.concatenate([source, destination], axis=0)
    idx = jnp.pad(idx, ((0, 0), (0, LP - L)))
    idx_flat = idx.reshape(-1)

    pooled = _sc_pool()(idx_flat, table)

    b1r = b1.reshape(1, H)
    b2r = b2.reshape(1, H)
    b3r = b3.reshape(1, H)
    g1r = g1.reshape(1, H)
    g2r = g2.reshape(1, H)
    be1r = be1.reshape(1, H)
    be2r = be2.reshape(1, H)
    w4r = W4.reshape(1, H)
    b4r = b4.reshape(1, 1)

    h1, st1 = pl.pallas_call(
        _k1_body,
        grid=(GRID,),
        in_specs=[
            pl.BlockSpec((RB, D), lambda i: (i, 0)),
            pl.BlockSpec((RB, D), lambda i: (i + GRID, 0)),
            pl.BlockSpec((RB, L), lambda i: (i, 0)),
            pl.BlockSpec((RB, L), lambda i: (i, 0)),
            _full_spec((2 * D, H)),
            _full_spec((1, H)),
        ],
        out_specs=[_row_spec(H), _full_spec((2, H))],
        out_shape=[
            jax.ShapeDtypeStruct((B, H), jnp.float32),
            jax.ShapeDtypeStruct((2, H), jnp.float32),
        ],
    )(pooled, pooled, source, destination, W1, b1r)

    h2, st2 = pl.pallas_call(
        _k2_body,
        grid=(GRID,),
        in_specs=[
            _row_spec(H),
            _full_spec((2, H)),
            _full_spec((1, H)),
            _full_spec((1, H)),
            _full_spec((H, H)),
            _full_spec((1, H)),
        ],
        out_specs=[_row_spec(H), _full_spec((2, H))],
        out_shape=[
            jax.ShapeDtypeStruct((B, H), jnp.float32),
            jax.ShapeDtypeStruct((2, H), jnp.float32),
        ],
    )(h1, st1, g1r, be1r, W2, b2r)

    out = pl.pallas_call(
        _k3_body,
        grid=(GRID,),
        in_specs=[
            _row_spec(H),
            _full_spec((2, H)),
            _full_spec((1, H)),
            _full_spec((1, H)),
            _full_spec((H, H)),
            _full_spec((1, H)),
            _full_spec((1, H)),
            _full_spec((1, 1)),
        ],
        out_specs=pl.BlockSpec((RB, 1), lambda i: (i, 0)),
        out_shape=jax.ShapeDtypeStruct((B, 1), jnp.float32),
    )(h2, st2, g2r, be2r, W3, b3r, w4r, b4r)

    return out.reshape(B)
